# baseline (speedup 1.0000x reference)
import functools

import jax
import jax.numpy as jnp
from jax import lax
from jax.experimental import pallas as pl
from jax.experimental.pallas import tpu as pltpu
from jax.experimental.pallas import tpu_sc as plsc

VOCAB = 100001
EMBED = 128
BATCH = 4096
SEQ = 200

NCH = 2
NSUB = 16
LANES = 16
ROWS_PER_TEC = BATCH // NSUB
GROUPS_PER_TEC = ROWS_PER_TEC // LANES
GRP_IDX = LANES * SEQ
TEC_IDX = ROWS_PER_TEC * SEQ
GRP_PER_CHUNK = 2
NCHUNK = GROUPS_PER_TEC // GRP_PER_CHUNK
CHUNK_IDX = GRP_PER_CHUNK * GRP_IDX

PROJ_BLK = 32768
HALF = PROJ_BLK // 2
NBLK = (VOCAB + PROJ_BLK - 1) // PROJ_BLK
WPAD = NBLK * HALF
LOG_BLK = PROJ_BLK.bit_length() - 1
LOG_HALF = HALF.bit_length() - 1

UNROLL = 8
STEPS = SEQ // UNROLL


def _round_bf16_bits(u):
    rounded = u + jnp.int32(0x7FFF) + ((u >> 16) & 1)
    return lax.shift_right_logical(rounded, 16)


def _proj_body(b_ref, w_ref, x_ref, o0_ref, o1_ref):
    x = x_ref[...]
    w = w_ref[...]
    p = lax.dot_general(w, x, (((1,), (1,)), ((), ())),
                        preferred_element_type=jnp.float32)
    inv = 1.0 / SEQ
    for c, o_ref in ((0, o0_ref), (1, o1_ref)):
        pc = p[c:c + 1, :] * inv + b_ref[c] * inv
        u = lax.bitcast_convert_type(pc, jnp.int32)
        word = (lax.shift_left(_round_bf16_bits(u[:, HALF:]), 16)
                | _round_bf16_bits(u[:, :HALF]))
        o_ref[...] = word[0]


def _project(fc_b, fc_w, emb):
    return pl.pallas_call(
        _proj_body,
        grid=(NBLK,),
        in_specs=[
            pl.BlockSpec(memory_space=pltpu.SMEM),
            pl.BlockSpec((NCH, EMBED), lambda i: (0, 0)),
            pl.BlockSpec((PROJ_BLK, EMBED), lambda i: (i, 0)),
        ],
        out_specs=[
            pl.BlockSpec((HALF,), lambda i: (i,)),
            pl.BlockSpec((HALF,), lambda i: (i,)),
        ],
        out_shape=[
            jax.ShapeDtypeStruct((WPAD,), jnp.int32),
            jax.ShapeDtypeStruct((WPAD,), jnp.int32),
        ],
    )(fc_b, fc_w, emb)


def _pool_body(p0_hbm, p1_hbm, idx_hbm, out_hbm, plane_v, idx_a, idx_b, out_v,
               sem_a, sem_b):
    ch = lax.axis_index("c")
    slot = lax.axis_index("s")
    base0 = slot * TEC_IDX
    bufs = [(idx_a, sem_a), (idx_b, sem_b)]

    def fire(c):
        buf, sem = bufs[c % 2]
        return pltpu.async_copy(
            idx_hbm.at[pl.ds(base0 + c * CHUNK_IDX, CHUNK_IDX)], buf, sem)

    pending = fire(0)

    @pl.when(ch == 0)
    def _():
        pltpu.sync_copy(p0_hbm, plane_v)

    @pl.when(ch == 1)
    def _():
        pltpu.sync_copy(p1_hbm, plane_v)

    lane_base = lax.iota(jnp.int32, LANES) * SEQ

    for c in range(NCHUNK):
        nxt = fire(c + 1) if c + 1 < NCHUNK else None
        pending.wait()
        buf = bufs[c % 2][0]
        for g in range(GRP_PER_CHUNK):
            base = g * GRP_IDX

            def step(i, acc, buf=buf, base=base):
                s0 = i * UNROLL
                vs = []
                for k in range(UNROLL):
                    rows = plsc.load_gather(
                        buf, [base + lane_base + (s0 + k)])
                    wi = (lax.shift_left(rows >> LOG_BLK, LOG_HALF)
                          | (rows & (HALF - 1)))
                    word = plsc.load_gather(plane_v, [wi])
                    sh = lax.shift_left((rows >> LOG_HALF) & 1, 4)
                    bits = lax.shift_left(
                        lax.shift_right_logical(word, sh), 16)
                    vs.append(plsc.bitcast(bits, jnp.float32))
                while len(vs) > 1:
                    vs = [a + b for a, b in zip(vs[::2], vs[1::2])]
                return acc + vs[0]

            acc = lax.fori_loop(0, STEPS, step,
                                jnp.zeros((LANES,), jnp.float32))
            out_v[pl.ds((c * GRP_PER_CHUNK + g) * LANES, LANES)] = acc
        pending = nxt

    pltpu.sync_copy(
        out_v,
        out_hbm.at[pl.ds(ch * BATCH + slot * ROWS_PER_TEC, ROWS_PER_TEC)])


@functools.cache
def _make_pool():
    return pl.kernel(
        _pool_body,
        mesh=plsc.VectorSubcoreMesh(core_axis_name="c", subcore_axis_name="s"),
        compiler_params=pltpu.CompilerParams(needs_layout_passes=False),
        out_type=jax.ShapeDtypeStruct((NCH * BATCH,), jnp.float32),
        scratch_types=[
            pltpu.VMEM((WPAD,), jnp.int32),
            pltpu.VMEM((CHUNK_IDX,), jnp.int32),
            pltpu.VMEM((CHUNK_IDX,), jnp.int32),
            pltpu.VMEM((ROWS_PER_TEC,), jnp.float32),
            pltpu.SemaphoreType.DMA,
            pltpu.SemaphoreType.DMA,
        ],
    )


def kernel(textid, emb, fc_w, fc_b):
    idx_flat = textid.reshape(-1).astype(jnp.int32)
    p0, p1 = _project(fc_b.astype(jnp.float32), fc_w.astype(jnp.float32),
                      emb.astype(jnp.float32))
    pooled = _make_pool()(p0, p1, idx_flat)
    return pooled.reshape(NCH, BATCH).T


# speedup vs baseline: 1.0417x; 1.0417x over previous
import functools

import jax
import jax.numpy as jnp
from jax import lax
from jax.experimental import pallas as pl
from jax.experimental.pallas import tpu as pltpu
from jax.experimental.pallas import tpu_sc as plsc

VOCAB = 100001
EMBED = 128
BATCH = 4096
SEQ = 200

NCH = 2
NSUB = 16
LANES = 16
ROWS_PER_TEC = BATCH // NSUB
GROUPS_PER_TEC = ROWS_PER_TEC // LANES
GRP_IDX = LANES * SEQ
TEC_IDX = ROWS_PER_TEC * SEQ
GRP_PER_CHUNK = 2
NCHUNK = GROUPS_PER_TEC // GRP_PER_CHUNK
CHUNK_IDX = GRP_PER_CHUNK * GRP_IDX

PROJ_BLK = 16384
HALF = PROJ_BLK // 2
NBLK = (VOCAB + PROJ_BLK - 1) // PROJ_BLK
WPAD = NBLK * HALF
LOG_BLK = PROJ_BLK.bit_length() - 1
LOG_HALF = HALF.bit_length() - 1

UNROLL = 8
STEPS = SEQ // UNROLL


def _round_bf16_bits(u):
    rounded = u + jnp.int32(0x7FFF) + ((u >> 16) & 1)
    return lax.shift_right_logical(rounded, 16)


def _proj_body(b_ref, w_ref, x_ref, o0_ref, o1_ref):
    x = x_ref[...]
    w = w_ref[...]
    p = lax.dot_general(w, x, (((1,), (1,)), ((), ())),
                        preferred_element_type=jnp.float32)
    inv = 1.0 / SEQ
    for c, o_ref in ((0, o0_ref), (1, o1_ref)):
        pc = p[c:c + 1, :] * inv + b_ref[c] * inv
        u = lax.bitcast_convert_type(pc, jnp.int32)
        word = (lax.shift_left(_round_bf16_bits(u[:, HALF:]), 16)
                | _round_bf16_bits(u[:, :HALF]))
        o_ref[...] = word[0]


def _project(fc_b, fc_w, emb):
    return pl.pallas_call(
        _proj_body,
        grid=(NBLK,),
        in_specs=[
            pl.BlockSpec(memory_space=pltpu.SMEM),
            pl.BlockSpec((NCH, EMBED), lambda i: (0, 0)),
            pl.BlockSpec((PROJ_BLK, EMBED), lambda i: (i, 0)),
        ],
        out_specs=[
            pl.BlockSpec((HALF,), lambda i: (i,)),
            pl.BlockSpec((HALF,), lambda i: (i,)),
        ],
        out_shape=[
            jax.ShapeDtypeStruct((WPAD,), jnp.int32),
            jax.ShapeDtypeStruct((WPAD,), jnp.int32),
        ],
    )(fc_b, fc_w, emb)


def _pool_body(p0_hbm, p1_hbm, idx_hbm, out_hbm, plane_v, idx_a, idx_b, out_v,
               sem_a, sem_b):
    ch = lax.axis_index("c")
    slot = lax.axis_index("s")
    base0 = slot * TEC_IDX
    bufs = [(idx_a, sem_a), (idx_b, sem_b)]

    def fire(c):
        buf, sem = bufs[c % 2]
        return pltpu.async_copy(
            idx_hbm.at[pl.ds(base0 + c * CHUNK_IDX, CHUNK_IDX)], buf, sem)

    pending = fire(0)

    @pl.when(ch == 0)
    def _():
        pltpu.sync_copy(p0_hbm, plane_v)

    @pl.when(ch == 1)
    def _():
        pltpu.sync_copy(p1_hbm, plane_v)

    lane_base = lax.iota(jnp.int32, LANES) * SEQ

    for c in range(NCHUNK):
        nxt = fire(c + 1) if c + 1 < NCHUNK else None
        pending.wait()
        buf = bufs[c % 2][0]
        for g in range(GRP_PER_CHUNK):
            base = g * GRP_IDX

            def step(i, acc, buf=buf, base=base):
                s0 = i * UNROLL
                vs = []
                for k in range(UNROLL):
                    rows = plsc.load_gather(
                        buf, [base + lane_base + (s0 + k)])
                    wi = (lax.shift_left(rows >> LOG_BLK, LOG_HALF)
                          | (rows & (HALF - 1)))
                    word = plsc.load_gather(plane_v, [wi])
                    sh = lax.shift_left((rows >> LOG_HALF) & 1, 4)
                    bits = lax.shift_left(
                        lax.shift_right_logical(word, sh), 16)
                    vs.append(plsc.bitcast(bits, jnp.float32))
                while len(vs) > 1:
                    vs = [a + b for a, b in zip(vs[::2], vs[1::2])]
                return acc + vs[0]

            acc = lax.fori_loop(0, STEPS, step,
                                jnp.zeros((LANES,), jnp.float32))
            out_v[pl.ds((c * GRP_PER_CHUNK + g) * LANES, LANES)] = acc
        pending = nxt

    pltpu.sync_copy(
        out_v,
        out_hbm.at[pl.ds(ch * BATCH + slot * ROWS_PER_TEC, ROWS_PER_TEC)])


@functools.cache
def _make_pool():
    return pl.kernel(
        _pool_body,
        mesh=plsc.VectorSubcoreMesh(core_axis_name="c", subcore_axis_name="s"),
        compiler_params=pltpu.CompilerParams(needs_layout_passes=False),
        out_type=jax.ShapeDtypeStruct((NCH * BATCH,), jnp.float32),
        scratch_types=[
            pltpu.VMEM((WPAD,), jnp.int32),
            pltpu.VMEM((CHUNK_IDX,), jnp.int32),
            pltpu.VMEM((CHUNK_IDX,), jnp.int32),
            pltpu.VMEM((ROWS_PER_TEC,), jnp.float32),
            pltpu.SemaphoreType.DMA,
            pltpu.SemaphoreType.DMA,
        ],
    )


def kernel(textid, emb, fc_w, fc_b):
    idx_flat = textid.reshape(-1).astype(jnp.int32)
    p0, p1 = _project(fc_b.astype(jnp.float32), fc_w.astype(jnp.float32),
                      emb.astype(jnp.float32))
    pooled = _make_pool()(p0, p1, idx_flat)
    return pooled.reshape(NCH, BATCH).T


# speedup vs baseline: 1.0742x; 1.0312x over previous
import functools

import jax
import jax.numpy as jnp
from jax import lax
from jax.experimental import pallas as pl
from jax.experimental.pallas import tpu as pltpu
from jax.experimental.pallas import tpu_sc as plsc

VOCAB = 100001
EMBED = 128
BATCH = 4096
SEQ = 200

NCH = 2
NSUB = 16
LANES = 16
ROWS_PER_TEC = BATCH // NSUB
GROUPS_PER_TEC = ROWS_PER_TEC // LANES
GRP_IDX = LANES * SEQ
TEC_IDX = ROWS_PER_TEC * SEQ
GRP_PER_CHUNK = 4
NCHUNK = GROUPS_PER_TEC // GRP_PER_CHUNK
CHUNK_IDX = GRP_PER_CHUNK * GRP_IDX

PROJ_BLK = 16384
HALF = PROJ_BLK // 2
NBLK = (VOCAB + PROJ_BLK - 1) // PROJ_BLK
WPAD = NBLK * HALF
LOG_BLK = PROJ_BLK.bit_length() - 1
LOG_HALF = HALF.bit_length() - 1

UNROLL = 10
STEPS = SEQ // UNROLL


def _round_bf16_bits(u):
    rounded = u + jnp.int32(0x7FFF) + ((u >> 16) & 1)
    return lax.shift_right_logical(rounded, 16)


def _proj_body(b_ref, w_ref, x_ref, o0_ref, o1_ref):
    x = x_ref[...]
    w = w_ref[...]
    p = lax.dot_general(w, x, (((1,), (1,)), ((), ())),
                        preferred_element_type=jnp.float32)
    inv = 1.0 / SEQ
    for c, o_ref in ((0, o0_ref), (1, o1_ref)):
        pc = p[c:c + 1, :] * inv + b_ref[c] * inv
        u = lax.bitcast_convert_type(pc, jnp.int32)
        word = (lax.shift_left(_round_bf16_bits(u[:, HALF:]), 16)
                | _round_bf16_bits(u[:, :HALF]))
        o_ref[...] = word[0]


def _project(fc_b, fc_w, emb):
    return pl.pallas_call(
        _proj_body,
        grid=(NBLK,),
        in_specs=[
            pl.BlockSpec(memory_space=pltpu.SMEM),
            pl.BlockSpec((NCH, EMBED), lambda i: (0, 0)),
            pl.BlockSpec((PROJ_BLK, EMBED), lambda i: (i, 0)),
        ],
        out_specs=[
            pl.BlockSpec((HALF,), lambda i: (i,)),
            pl.BlockSpec((HALF,), lambda i: (i,)),
        ],
        out_shape=[
            jax.ShapeDtypeStruct((WPAD,), jnp.int32),
            jax.ShapeDtypeStruct((WPAD,), jnp.int32),
        ],
    )(fc_b, fc_w, emb)


def _pool_body(p0_hbm, p1_hbm, idx_hbm, out_hbm, plane_v, idx_a, idx_b, out_v,
               sem_a, sem_b):
    ch = lax.axis_index("c")
    slot = lax.axis_index("s")
    base0 = slot * TEC_IDX
    bufs = [(idx_a, sem_a), (idx_b, sem_b)]

    def fire(c):
        buf, sem = bufs[c % 2]
        return pltpu.async_copy(
            idx_hbm.at[pl.ds(base0 + c * CHUNK_IDX, CHUNK_IDX)], buf, sem)

    pending = fire(0)

    @pl.when(ch == 0)
    def _():
        pltpu.sync_copy(p0_hbm, plane_v)

    @pl.when(ch == 1)
    def _():
        pltpu.sync_copy(p1_hbm, plane_v)

    lane_base = lax.iota(jnp.int32, LANES) * SEQ

    for c in range(NCHUNK):
        nxt = fire(c + 1) if c + 1 < NCHUNK else None
        pending.wait()
        buf = bufs[c % 2][0]
        for g in range(GRP_PER_CHUNK):
            base = g * GRP_IDX

            def step(i, acc, buf=buf, base=base):
                s0 = i * UNROLL
                vs = []
                for k in range(UNROLL):
                    rows = plsc.load_gather(
                        buf, [base + lane_base + (s0 + k)])
                    wi = (lax.shift_left(rows >> LOG_BLK, LOG_HALF)
                          | (rows & (HALF - 1)))
                    word = plsc.load_gather(plane_v, [wi])
                    sh = lax.shift_left((rows >> LOG_HALF) & 1, 4)
                    bits = lax.shift_left(
                        lax.shift_right_logical(word, sh), 16)
                    vs.append(plsc.bitcast(bits, jnp.float32))
                while len(vs) > 1:
                    vs = [a + b for a, b in zip(vs[::2], vs[1::2])]
                return acc + vs[0]

            acc = lax.fori_loop(0, STEPS, step,
                                jnp.zeros((LANES,), jnp.float32))
            out_v[pl.ds((c * GRP_PER_CHUNK + g) * LANES, LANES)] = acc
        pending = nxt

    pltpu.sync_copy(
        out_v,
        out_hbm.at[pl.ds(ch * BATCH + slot * ROWS_PER_TEC, ROWS_PER_TEC)])


@functools.cache
def _make_pool():
    return pl.kernel(
        _pool_body,
        mesh=plsc.VectorSubcoreMesh(core_axis_name="c", subcore_axis_name="s"),
        compiler_params=pltpu.CompilerParams(needs_layout_passes=False),
        out_type=jax.ShapeDtypeStruct((NCH * BATCH,), jnp.float32),
        scratch_types=[
            pltpu.VMEM((WPAD,), jnp.int32),
            pltpu.VMEM((CHUNK_IDX,), jnp.int32),
            pltpu.VMEM((CHUNK_IDX,), jnp.int32),
            pltpu.VMEM((ROWS_PER_TEC,), jnp.float32),
            pltpu.SemaphoreType.DMA,
            pltpu.SemaphoreType.DMA,
        ],
    )


def kernel(textid, emb, fc_w, fc_b):
    idx_flat = textid.reshape(-1).astype(jnp.int32)
    p0, p1 = _project(fc_b.astype(jnp.float32), fc_w.astype(jnp.float32),
                      emb.astype(jnp.float32))
    pooled = _make_pool()(p0, p1, idx_flat)
    return pooled.reshape(NCH, BATCH).T
